# Initial kernel scaffold; baseline (speedup 1.0000x reference)
import jax
import jax.numpy as jnp
from jax.experimental import pallas as pl


def kernel(x, edge_index, W1_v2e, W1_e2v, bn1_gamma, bn1_beta, W2_v2e, W2_e2v):
    raise NotImplementedError("write your pallas kernel here")


# speedup vs baseline: 4.3882x; 4.3882x over previous
import functools

import jax
import jax.numpy as jnp
import numpy as np
from jax import lax
from jax.experimental import pallas as pl
from jax.experimental.pallas import tpu as pltpu
from jax.experimental.pallas import tpu_sc as plsc

_N = 10000
_E = 320000
_NPAD = 10240
_NC = 2
_NS = 16
_NW = _NC * _NS
_PW = _E // _NW
_K = 80
_CH = _PW // _K
_RSUB = _NPAD // _NS
_R = 512
_BN_SCALE = float(1.0 / np.sqrt(1.0 + 1e-5))

_mesh = plsc.VectorSubcoreMesh(
    core_axis_name="c", subcore_axis_name="s", num_cores=_NC, num_subcores=_NS)


def _make_sc_segsum(C):
    @functools.partial(
        pl.kernel,
        out_type=jax.ShapeDtypeStruct((_NC, _NPAD, C), jnp.float32),
        mesh=_mesh,
        compiler_params=pltpu.CompilerParams(use_tc_tiling_on_sc=False),
        scratch_types=[pltpu.VMEM((_K,), jnp.int32),
                       pltpu.VMEM((_K,), jnp.int32),
                       pltpu.VMEM((_K, C), jnp.float32),
                       pltpu.VMEM_SHARED((_NPAD, C), jnp.float32),
                       pltpu.SemaphoreType.DMA])
    def _seg(table_hbm, gidx_hbm, sidx_hbm, zrows_hbm, out_hbm,
             gbuf, sbuf, rows, acc, sem):
        cid = lax.axis_index("c")
        sid = lax.axis_index("s")
        wid = sid * _NC + cid
        row0 = pl.multiple_of(sid * _RSUB, 8)
        pltpu.sync_copy(zrows_hbm, acc.at[pl.ds(row0, _RSUB)])
        plsc.subcore_barrier()

        base = pl.multiple_of(wid * _PW, 8)

        def chunk_body(cI, carry):
            cb = pl.multiple_of(base + cI * _K, 8)
            pltpu.sync_copy(gidx_hbm.at[pl.ds(cb, _K)], gbuf)
            pltpu.sync_copy(sidx_hbm.at[pl.ds(cb, _K)], sbuf)
            pltpu.async_copy(table_hbm.at[gbuf], rows, sem).wait()
            pltpu.sync_copy(rows, acc.at[sbuf], add=True)
            return carry
        lax.fori_loop(0, _CH, chunk_body, 0)

        plsc.subcore_barrier()
        pltpu.sync_copy(acc.at[pl.ds(row0, _RSUB)],
                        out_hbm.at[cid, pl.ds(row0, _RSUB)])

    return _seg


_sc_seg144 = _make_sc_segsum(144)
_sc_seg64 = _make_sc_segsum(64)


def _ones_tail(r):
    col = lax.broadcasted_iota(jnp.int32, (r, 16), 1)
    return jnp.where(col == 0, 1.0, 0.0).astype(jnp.float32)


def _tc1_body(x_ref, w_ref, o_ref):
    d = jnp.dot(x_ref[...], w_ref[...], preferred_element_type=jnp.float32)
    o_ref[...] = jnp.concatenate([d, _ones_tail(_R)], axis=1)


def _tc2_body(p_ref, w_ref, o_ref, c_out):
    s = p_ref[0] + p_ref[1]
    cnt = jnp.sum(s[:, 128:144], axis=1, keepdims=True)
    c_out[...] = cnt
    yc = jnp.maximum(s[:, :128] / jnp.maximum(cnt, 1.0), 0.0)
    d = jnp.dot(yc, w_ref[...], preferred_element_type=jnp.float32)
    o_ref[...] = jnp.concatenate([d, _ones_tail(_R)], axis=1)


def _tc3_body(p_ref, g_ref, b_ref, w_ref, o_ref, c_out):
    s = p_ref[0] + p_ref[1]
    cnt = jnp.sum(s[:, 128:144], axis=1, keepdims=True)
    c_out[...] = cnt
    xv = jnp.maximum(s[:, :128] / jnp.maximum(cnt, 1.0), 0.0)
    xv = xv * (g_ref[...] * _BN_SCALE) + b_ref[...]
    o_ref[...] = jnp.dot(xv, w_ref[...], preferred_element_type=jnp.float32)


def _tc4_body(p_ref, c_ref, w_ref, o_ref):
    s = p_ref[0] + p_ref[1]
    cnt = c_ref[...]
    yc = jnp.maximum(s / jnp.maximum(cnt, 1.0), 0.0)
    o_ref[...] = jnp.dot(yc, w_ref[...], preferred_element_type=jnp.float32)


def _tc5_body(p_ref, c_ref, o_ref):
    s = p_ref[0] + p_ref[1]
    cnt = c_ref[...]
    z = s / jnp.maximum(cnt, 1.0)
    col = lax.broadcasted_iota(jnp.int32, (_R, 64), 1)
    valid = col < 40
    zm = jnp.where(valid, z, -jnp.inf)
    m = jnp.max(zm, axis=1, keepdims=True)
    ez = jnp.where(valid, jnp.exp(z - m), 0.0)
    lse = jnp.log(jnp.sum(ez, axis=1, keepdims=True))
    o_ref[...] = jnp.where(valid, z - m - lse, 0.0)


_GRID = (_NPAD // _R,)


def _pblk(c):
    return pl.BlockSpec((_NC, _R, c), lambda i: (0, i, 0))


def _cblk():
    return pl.BlockSpec((_R, 1), lambda i: (i, 0))


def _wblk(cin, cout):
    return pl.BlockSpec((cin, cout), lambda i: (0, 0))


def _oblk(c):
    return pl.BlockSpec((_R, c), lambda i: (i, 0))


def _tc_matmul_ones(xp, w):
    return pl.pallas_call(
        _tc1_body, grid=_GRID,
        in_specs=[pl.BlockSpec((_R, 128), lambda i: (i, 0)), _wblk(128, 128)],
        out_specs=_oblk(144),
        out_shape=jax.ShapeDtypeStruct((_NPAD, 144), jnp.float32),
    )(xp, w)


def _tc_mean_relu_mm_ones(p, w):
    return pl.pallas_call(
        _tc2_body, grid=_GRID,
        in_specs=[_pblk(144), _wblk(128, 128)],
        out_specs=[_oblk(144), _cblk()],
        out_shape=[jax.ShapeDtypeStruct((_NPAD, 144), jnp.float32),
                   jax.ShapeDtypeStruct((_NPAD, 1), jnp.float32)],
    )(p, w)


def _tc_mean_relu_bn_mm(p, gam, bet, w):
    return pl.pallas_call(
        _tc3_body, grid=_GRID,
        in_specs=[_pblk(144),
                  pl.BlockSpec((1, 128), lambda i: (0, 0)),
                  pl.BlockSpec((1, 128), lambda i: (0, 0)),
                  _wblk(128, 64)],
        out_specs=[_oblk(64), _cblk()],
        out_shape=[jax.ShapeDtypeStruct((_NPAD, 64), jnp.float32),
                   jax.ShapeDtypeStruct((_NPAD, 1), jnp.float32)],
    )(p, gam, bet, w)


def _tc_mean_relu_mm64(p, cp, w):
    return pl.pallas_call(
        _tc4_body, grid=_GRID,
        in_specs=[pl.BlockSpec((_NC, _R, 64), lambda i: (0, i, 0)),
                  _cblk(), _wblk(64, 64)],
        out_specs=_oblk(64),
        out_shape=jax.ShapeDtypeStruct((_NPAD, 64), jnp.float32),
    )(p, cp, w)


def _tc_mean_logsoftmax(p, cp):
    return pl.pallas_call(
        _tc5_body, grid=_GRID,
        in_specs=[pl.BlockSpec((_NC, _R, 64), lambda i: (0, i, 0)), _cblk()],
        out_specs=_oblk(64),
        out_shape=jax.ShapeDtypeStruct((_NPAD, 64), jnp.float32),
    )(p, cp)


def kernel(x, edge_index, W1_v2e, W1_e2v, bn1_gamma, bn1_beta, W2_v2e, W2_e2v):
    f32 = jnp.float32
    vidx = edge_index[0]
    eidx = edge_index[1]
    xp = jnp.zeros((_NPAD, 128), f32).at[:_N, :].set(x)
    w2v = jnp.zeros((128, 64), f32).at[:, :40].set(W2_v2e)
    w2e = jnp.zeros((64, 64), f32).at[:40, :40].set(W2_e2v)
    gam = bn1_gamma.reshape(1, 128)
    bet = bn1_beta.reshape(1, 128)
    z144 = jnp.zeros((_RSUB, 144), f32)
    z64 = jnp.zeros((_RSUB, 64), f32)

    x1 = _tc_matmul_ones(xp, W1_v2e)
    p1 = _sc_seg144(x1, vidx, eidx, z144)
    y1, ce = _tc_mean_relu_mm_ones(p1, W1_e2v)
    p2 = _sc_seg144(y1, eidx, vidx, z144)
    x2, cv = _tc_mean_relu_bn_mm(p2, gam, bet, w2v)
    p3 = _sc_seg64(x2, vidx, eidx, z64)
    y2 = _tc_mean_relu_mm64(p3, ce, w2e)
    p4 = _sc_seg64(y2, eidx, vidx, z64)
    out = _tc_mean_logsoftmax(p4, cv)
    return out[:_N, :40]


# speedup vs baseline: 4.4367x; 1.0110x over previous
import functools

import jax
import jax.numpy as jnp
import numpy as np
from jax import lax
from jax.experimental import pallas as pl
from jax.experimental.pallas import tpu as pltpu
from jax.experimental.pallas import tpu_sc as plsc

_N = 10000
_E = 320000
_NPAD = 10240
_NC = 2
_NS = 16
_K = 128
_CHP = 160
_EP = _NS * _CHP * _K
_TRASH = 10200
_RSUB = _NPAD // _NS
_R = 512
_BN_SCALE = float(1.0 / np.sqrt(1.0 + 1e-5))

_mesh = plsc.VectorSubcoreMesh(
    core_axis_name="c", subcore_axis_name="s", num_cores=_NC, num_subcores=_NS)


def _make_sc_segsum(W):
    @functools.partial(
        pl.kernel,
        out_type=[jax.ShapeDtypeStruct((_NPAD, W), jnp.float32),
                  jax.ShapeDtypeStruct((_NPAD, W), jnp.float32)],
        mesh=_mesh,
        compiler_params=pltpu.CompilerParams(use_tc_tiling_on_sc=False),
        scratch_types=[pltpu.VMEM((_CHP, _K), jnp.int32),
                       pltpu.VMEM((_CHP, _K), jnp.int32),
                       pltpu.VMEM((_K, W), jnp.float32),
                       pltpu.VMEM((_K, W), jnp.float32),
                       pltpu.VMEM_SHARED((_NPAD, W), jnp.float32),
                       pltpu.SemaphoreType.DMA,
                       pltpu.SemaphoreType.DMA,
                       pltpu.SemaphoreType.DMA])
    def _seg(tl_hbm, tr_hbm, gidx_hbm, sidx_hbm, zrows_hbm, outl_hbm,
             outr_hbm, gbuf, sbuf, rows0, rows1, acc, sem0, sem1, zsem):
        cid = lax.axis_index("c")
        sid = lax.axis_index("s")
        row0 = pl.multiple_of(sid * _RSUB, 8)
        zcopy = pltpu.async_copy(zrows_hbm, acc.at[pl.ds(row0, _RSUB)], zsem)
        pltpu.sync_copy(gidx_hbm.at[sid], gbuf)
        pltpu.sync_copy(sidx_hbm.at[sid], sbuf)
        zcopy.wait()
        plsc.subcore_barrier()

        def run(table_hbm, out_hbm):
            pltpu.async_copy(table_hbm.at[gbuf.at[0]], rows0, sem0)

            def chunk_pair(i2, carry):
                j0 = i2 * 2
                pltpu.async_copy(table_hbm.at[gbuf.at[j0 + 1]], rows1, sem1)
                pltpu.make_async_copy(table_hbm.at[gbuf.at[j0]], rows0,
                                      sem0).wait()
                pltpu.sync_copy(rows0, acc.at[sbuf.at[j0]], add=True)

                @pl.when(i2 < _CHP // 2 - 1)
                def _():
                    pltpu.async_copy(table_hbm.at[gbuf.at[j0 + 2]], rows0,
                                     sem0)
                pltpu.make_async_copy(table_hbm.at[gbuf.at[j0 + 1]], rows1,
                                      sem1).wait()
                pltpu.sync_copy(rows1, acc.at[sbuf.at[j0 + 1]], add=True)
                return carry
            lax.fori_loop(0, _CHP // 2, chunk_pair, 0)

            plsc.subcore_barrier()
            pltpu.sync_copy(acc.at[pl.ds(row0, _RSUB)],
                            out_hbm.at[pl.ds(row0, _RSUB)])

        @pl.when(cid == 0)
        def _():
            run(tl_hbm, outl_hbm)

        @pl.when(cid == 1)
        def _():
            run(tr_hbm, outr_hbm)

    return _seg


_sc_seg80 = _make_sc_segsum(80)
_sc_seg32 = _make_sc_segsum(32)


def _ones_tail(r):
    col = lax.broadcasted_iota(jnp.int32, (r, 16), 1)
    return jnp.where(col == 0, 1.0, 0.0).astype(jnp.float32)


def _split_ones(d):
    zs = jnp.zeros((_R, 16), jnp.float32)
    return (jnp.concatenate([d[:, :64], _ones_tail(_R)], axis=1),
            jnp.concatenate([d[:, 64:128], zs], axis=1))


def _tc1_body(x_ref, w_ref, ol_ref, or_ref):
    d = jnp.dot(x_ref[...], w_ref[...], preferred_element_type=jnp.float32)
    ol_ref[...], or_ref[...] = _split_ones(d)


def _mean128(pl_ref, pr_ref):
    sl = pl_ref[...]
    cnt = jnp.sum(sl[:, 64:80], axis=1, keepdims=True)
    s = jnp.concatenate([sl[:, :64], pr_ref[...][:, :64]], axis=1)
    return s / jnp.maximum(cnt, 1.0), cnt


def _tc2_body(pl_ref, pr_ref, w_ref, ol_ref, or_ref, c_out):
    m, cnt = _mean128(pl_ref, pr_ref)
    c_out[...] = cnt
    yc = jnp.maximum(m, 0.0)
    d = jnp.dot(yc, w_ref[...], preferred_element_type=jnp.float32)
    ol_ref[...], or_ref[...] = _split_ones(d)


def _tc3_body(pl_ref, pr_ref, g_ref, b_ref, w_ref, ol_ref, or_ref, c_out):
    m, cnt = _mean128(pl_ref, pr_ref)
    c_out[...] = cnt
    xv = jnp.maximum(m, 0.0)
    xv = xv * (g_ref[...] * _BN_SCALE) + b_ref[...]
    d = jnp.dot(xv, w_ref[...], preferred_element_type=jnp.float32)
    ol_ref[...] = d[:, :32]
    or_ref[...] = d[:, 32:64]


def _tc4_body(rl_ref, rr_ref, c_ref, w_ref, ol_ref, or_ref):
    s = jnp.concatenate([rl_ref[...], rr_ref[...]], axis=1)
    yc = jnp.maximum(s / jnp.maximum(c_ref[...], 1.0), 0.0)
    d = jnp.dot(yc, w_ref[...], preferred_element_type=jnp.float32)
    ol_ref[...] = d[:, :32]
    or_ref[...] = d[:, 32:64]


def _tc5_body(rl_ref, rr_ref, c_ref, o_ref):
    s = jnp.concatenate([rl_ref[...], rr_ref[...]], axis=1)
    z = s / jnp.maximum(c_ref[...], 1.0)
    col = lax.broadcasted_iota(jnp.int32, (_R, 64), 1)
    valid = col < 40
    zm = jnp.where(valid, z, -jnp.inf)
    m = jnp.max(zm, axis=1, keepdims=True)
    ez = jnp.where(valid, jnp.exp(z - m), 0.0)
    lse = jnp.log(jnp.sum(ez, axis=1, keepdims=True))
    o_ref[...] = jnp.where(valid, z - m - lse, 0.0)


_GRID = (_NPAD // _R,)


def _blk(c):
    return pl.BlockSpec((_R, c), lambda i: (i, 0))


def _cblk():
    return pl.BlockSpec((_R, 1), lambda i: (i, 0))


def _wblk(cin, cout):
    return pl.BlockSpec((cin, cout), lambda i: (0, 0))


def _f32(*shape):
    return jax.ShapeDtypeStruct(shape, jnp.float32)


def _tc_matmul_ones(xp, w):
    return pl.pallas_call(
        _tc1_body, grid=_GRID,
        in_specs=[_blk(128), _wblk(128, 128)],
        out_specs=[_blk(80), _blk(80)],
        out_shape=[_f32(_NPAD, 80), _f32(_NPAD, 80)],
    )(xp, w)


def _tc_mean_relu_mm_ones(pL, pR, w):
    return pl.pallas_call(
        _tc2_body, grid=_GRID,
        in_specs=[_blk(80), _blk(80), _wblk(128, 128)],
        out_specs=[_blk(80), _blk(80), _cblk()],
        out_shape=[_f32(_NPAD, 80), _f32(_NPAD, 80), _f32(_NPAD, 1)],
    )(pL, pR, w)


def _tc_mean_relu_bn_mm(pL, pR, gam, bet, w):
    return pl.pallas_call(
        _tc3_body, grid=_GRID,
        in_specs=[_blk(80), _blk(80),
                  pl.BlockSpec((1, 128), lambda i: (0, 0)),
                  pl.BlockSpec((1, 128), lambda i: (0, 0)),
                  _wblk(128, 64)],
        out_specs=[_blk(32), _blk(32), _cblk()],
        out_shape=[_f32(_NPAD, 32), _f32(_NPAD, 32), _f32(_NPAD, 1)],
    )(pL, pR, gam, bet, w)


def _tc_mean_relu_mm64(rL, rR, cnt, w):
    return pl.pallas_call(
        _tc4_body, grid=_GRID,
        in_specs=[_blk(32), _blk(32), _cblk(), _wblk(64, 64)],
        out_specs=[_blk(32), _blk(32)],
        out_shape=[_f32(_NPAD, 32), _f32(_NPAD, 32)],
    )(rL, rR, cnt, w)


def _tc_mean_logsoftmax(rL, rR, cnt):
    return pl.pallas_call(
        _tc5_body, grid=_GRID,
        in_specs=[_blk(32), _blk(32), _cblk()],
        out_specs=_blk(64),
        out_shape=_f32(_NPAD, 64),
    )(rL, rR, cnt)


def kernel(x, edge_index, W1_v2e, W1_e2v, bn1_gamma, bn1_beta, W2_v2e, W2_e2v):
    f32 = jnp.float32
    i32 = jnp.int32
    vidx = edge_index[0]
    eidx = edge_index[1]
    gpad = jnp.zeros((_EP - _E,), i32)
    spad = jnp.full((_EP - _E,), _TRASH, i32)
    v3 = jnp.concatenate([vidx, gpad]).reshape(_NS, _CHP, _K)
    e3 = jnp.concatenate([eidx, gpad]).reshape(_NS, _CHP, _K)
    vs3 = jnp.concatenate([vidx, spad]).reshape(_NS, _CHP, _K)
    es3 = jnp.concatenate([eidx, spad]).reshape(_NS, _CHP, _K)
    xp = jnp.zeros((_NPAD, 128), f32).at[:_N, :].set(x)
    w2v = jnp.zeros((128, 64), f32).at[:, :40].set(W2_v2e)
    w2e = jnp.zeros((64, 64), f32).at[:40, :40].set(W2_e2v)
    gam = bn1_gamma.reshape(1, 128)
    bet = bn1_beta.reshape(1, 128)
    z80 = jnp.zeros((_RSUB, 80), f32)
    z32 = jnp.zeros((_RSUB, 32), f32)

    tL, tR = _tc_matmul_ones(xp, W1_v2e)
    p1L, p1R = _sc_seg80(tL, tR, v3, es3, z80)
    yL, yR, ce = _tc_mean_relu_mm_ones(p1L, p1R, W1_e2v)
    p2L, p2R = _sc_seg80(yL, yR, e3, vs3, z80)
    qL, qR, cv = _tc_mean_relu_bn_mm(p2L, p2R, gam, bet, w2v)
    r1L, r1R = _sc_seg32(qL, qR, v3, es3, z32)
    sL, sR = _tc_mean_relu_mm64(r1L, r1R, ce, w2e)
    r2L, r2R = _sc_seg32(sL, sR, e3, vs3, z32)
    out = _tc_mean_logsoftmax(r2L, r2R, cv)
    return out[:_N, :40]


# speedup vs baseline: 4.7072x; 1.0610x over previous
import functools

import jax
import jax.numpy as jnp
import numpy as np
from jax import lax
from jax.experimental import pallas as pl
from jax.experimental.pallas import tpu as pltpu
from jax.experimental.pallas import tpu_sc as plsc

_N = 10000
_E = 320000
_NPAD = 10240
_NC = 2
_NS = 16
_K = 128
_CHP = 160
_CHH = _CHP // 2
_EP = _NS * _CHP * _K
_TRASH = 10200
_RSUB = _NPAD // _NS
_R = 512
_BN_SCALE = float(1.0 / np.sqrt(1.0 + 1e-5))

_mesh = plsc.VectorSubcoreMesh(
    core_axis_name="c", subcore_axis_name="s", num_cores=_NC, num_subcores=_NS)


def _make_sc_segsum(W):
    @functools.partial(
        pl.kernel,
        out_type=[jax.ShapeDtypeStruct((_NPAD, W), jnp.float32),
                  jax.ShapeDtypeStruct((_NPAD, W), jnp.float32)],
        mesh=_mesh,
        compiler_params=pltpu.CompilerParams(use_tc_tiling_on_sc=False),
        scratch_types=[pltpu.VMEM((_CHH, _K), jnp.int32),
                       pltpu.VMEM((_CHH, _K), jnp.int32),
                       pltpu.VMEM((_K, W), jnp.float32),
                       pltpu.VMEM((_K, W), jnp.float32),
                       pltpu.VMEM((_K, W), jnp.float32),
                       pltpu.VMEM((_K, W), jnp.float32),
                       pltpu.VMEM_SHARED((_NPAD, W), jnp.float32),
                       pltpu.SemaphoreType.DMA,
                       pltpu.SemaphoreType.DMA,
                       pltpu.SemaphoreType.DMA,
                       pltpu.SemaphoreType.DMA,
                       pltpu.SemaphoreType.DMA,
                       pltpu.SemaphoreType.DMA,
                       pltpu.SemaphoreType.DMA,
                       pltpu.SemaphoreType.DMA,
                       pltpu.SemaphoreType.DMA])
    def _seg(tl_hbm, tr_hbm, gidx_hbm, sidx_hbm, zrows_hbm, outl_hbm,
             outr_hbm, gbuf, sbuf, r0, r1, r2, r3,
             acc, g0, g1, g2, g3, s0, s1, s2, s3, zsem):
        cid = lax.axis_index("c")
        sid = lax.axis_index("s")
        rows = [r0, r1, r2, r3]
        gsem = [g0, g1, g2, g3]
        ssem = [s0, s1, s2, s3]
        row0 = pl.multiple_of(sid * _RSUB, 8)
        zcopy = pltpu.async_copy(zrows_hbm, acc.at[pl.ds(row0, _RSUB)], zsem)
        zcopy.wait()
        plsc.subcore_barrier()

        def run(table_hbm, out_hbm):
            def gather_start(j, b):
                pltpu.async_copy(table_hbm.at[gbuf.at[j]], rows[b], gsem[b])

            def gather_wait(j, b):
                pltpu.make_async_copy(table_hbm.at[gbuf.at[j]], rows[b],
                                      gsem[b]).wait()

            def scatter_start(j, b):
                pltpu.async_copy(rows[b], acc.at[sbuf.at[j]], ssem[b],
                                 add=True)

            def scatter_wait(j, b):
                pltpu.make_async_copy(rows[b], acc.at[sbuf.at[j]],
                                      ssem[b]).wait()

            for h in range(2):
                pltpu.sync_copy(gidx_hbm.at[sid, pl.ds(h * _CHH, _CHH)], gbuf)
                pltpu.sync_copy(sidx_hbm.at[sid, pl.ds(h * _CHH, _CHH)], sbuf)
                gather_start(0, 0)
                gather_start(1, 1)
                gather_start(2, 2)
                gather_start(3, 3)

                def quad(i, carry):
                    for b in range(4):
                        j = i * 4 + b
                        bn = (b + 2) % 4
                        if b < 2:
                            @pl.when(i > 0)
                            def _():
                                scatter_wait(j - 2, bn)
                                gather_start(j + 2, bn)
                        else:
                            scatter_wait(j - 2, bn)

                            @pl.when(i < _CHH // 4 - 1)
                            def _():
                                gather_start(j + 2, bn)
                        gather_wait(j, b)
                        scatter_start(j, b)
                    return carry
                lax.fori_loop(0, _CHH // 4, quad, 0)
                scatter_wait(_CHH - 2, (_CHH - 2) % 4)
                scatter_wait(_CHH - 1, (_CHH - 1) % 4)

            plsc.subcore_barrier()
            pltpu.sync_copy(acc.at[pl.ds(row0, _RSUB)],
                            out_hbm.at[pl.ds(row0, _RSUB)])

        @pl.when(cid == 0)
        def _():
            run(tl_hbm, outl_hbm)

        @pl.when(cid == 1)
        def _():
            run(tr_hbm, outr_hbm)

    return _seg


_sc_seg80 = _make_sc_segsum(80)
_sc_seg32 = _make_sc_segsum(32)


def _ones_tail(r):
    col = lax.broadcasted_iota(jnp.int32, (r, 16), 1)
    return jnp.where(col == 0, 1.0, 0.0).astype(jnp.float32)


def _split_ones(d):
    zs = jnp.zeros((_R, 16), jnp.float32)
    return (jnp.concatenate([d[:, :64], _ones_tail(_R)], axis=1),
            jnp.concatenate([d[:, 64:128], zs], axis=1))


def _tc1_body(x_ref, w_ref, ol_ref, or_ref):
    d = jnp.dot(x_ref[...], w_ref[...], preferred_element_type=jnp.float32)
    ol_ref[...], or_ref[...] = _split_ones(d)


def _mean128(pl_ref, pr_ref):
    sl = pl_ref[...]
    cnt = jnp.sum(sl[:, 64:80], axis=1, keepdims=True)
    s = jnp.concatenate([sl[:, :64], pr_ref[...][:, :64]], axis=1)
    return s / jnp.maximum(cnt, 1.0), cnt


def _tc2_body(pl_ref, pr_ref, w_ref, ol_ref, or_ref, c_out):
    m, cnt = _mean128(pl_ref, pr_ref)
    c_out[...] = cnt
    yc = jnp.maximum(m, 0.0)
    d = jnp.dot(yc, w_ref[...], preferred_element_type=jnp.float32)
    ol_ref[...], or_ref[...] = _split_ones(d)


def _tc3_body(pl_ref, pr_ref, g_ref, b_ref, w_ref, ol_ref, or_ref, c_out):
    m, cnt = _mean128(pl_ref, pr_ref)
    c_out[...] = cnt
    xv = jnp.maximum(m, 0.0)
    xv = xv * (g_ref[...] * _BN_SCALE) + b_ref[...]
    d = jnp.dot(xv, w_ref[...], preferred_element_type=jnp.float32)
    ol_ref[...] = d[:, :32]
    or_ref[...] = d[:, 32:64]


def _tc4_body(rl_ref, rr_ref, c_ref, w_ref, ol_ref, or_ref):
    s = jnp.concatenate([rl_ref[...], rr_ref[...]], axis=1)
    yc = jnp.maximum(s / jnp.maximum(c_ref[...], 1.0), 0.0)
    d = jnp.dot(yc, w_ref[...], preferred_element_type=jnp.float32)
    ol_ref[...] = d[:, :32]
    or_ref[...] = d[:, 32:64]


def _tc5_body(rl_ref, rr_ref, c_ref, o_ref):
    s = jnp.concatenate([rl_ref[...], rr_ref[...]], axis=1)
    z = s / jnp.maximum(c_ref[...], 1.0)
    col = lax.broadcasted_iota(jnp.int32, (_R, 64), 1)
    valid = col < 40
    zm = jnp.where(valid, z, -jnp.inf)
    m = jnp.max(zm, axis=1, keepdims=True)
    ez = jnp.where(valid, jnp.exp(z - m), 0.0)
    lse = jnp.log(jnp.sum(ez, axis=1, keepdims=True))
    o_ref[...] = jnp.where(valid, z - m - lse, 0.0)


_GRID = (_NPAD // _R,)


def _blk(c):
    return pl.BlockSpec((_R, c), lambda i: (i, 0))


def _cblk():
    return pl.BlockSpec((_R, 1), lambda i: (i, 0))


def _wblk(cin, cout):
    return pl.BlockSpec((cin, cout), lambda i: (0, 0))


def _f32(*shape):
    return jax.ShapeDtypeStruct(shape, jnp.float32)


def _tc_matmul_ones(xp, w):
    return pl.pallas_call(
        _tc1_body, grid=_GRID,
        in_specs=[_blk(128), _wblk(128, 128)],
        out_specs=[_blk(80), _blk(80)],
        out_shape=[_f32(_NPAD, 80), _f32(_NPAD, 80)],
    )(xp, w)


def _tc_mean_relu_mm_ones(pL, pR, w):
    return pl.pallas_call(
        _tc2_body, grid=_GRID,
        in_specs=[_blk(80), _blk(80), _wblk(128, 128)],
        out_specs=[_blk(80), _blk(80), _cblk()],
        out_shape=[_f32(_NPAD, 80), _f32(_NPAD, 80), _f32(_NPAD, 1)],
    )(pL, pR, w)


def _tc_mean_relu_bn_mm(pL, pR, gam, bet, w):
    return pl.pallas_call(
        _tc3_body, grid=_GRID,
        in_specs=[_blk(80), _blk(80),
                  pl.BlockSpec((1, 128), lambda i: (0, 0)),
                  pl.BlockSpec((1, 128), lambda i: (0, 0)),
                  _wblk(128, 64)],
        out_specs=[_blk(32), _blk(32), _cblk()],
        out_shape=[_f32(_NPAD, 32), _f32(_NPAD, 32), _f32(_NPAD, 1)],
    )(pL, pR, gam, bet, w)


def _tc_mean_relu_mm64(rL, rR, cnt, w):
    return pl.pallas_call(
        _tc4_body, grid=_GRID,
        in_specs=[_blk(32), _blk(32), _cblk(), _wblk(64, 64)],
        out_specs=[_blk(32), _blk(32)],
        out_shape=[_f32(_NPAD, 32), _f32(_NPAD, 32)],
    )(rL, rR, cnt, w)


def _tc_mean_logsoftmax(rL, rR, cnt):
    return pl.pallas_call(
        _tc5_body, grid=_GRID,
        in_specs=[_blk(32), _blk(32), _cblk()],
        out_specs=_blk(64),
        out_shape=_f32(_NPAD, 64),
    )(rL, rR, cnt)


def kernel(x, edge_index, W1_v2e, W1_e2v, bn1_gamma, bn1_beta, W2_v2e, W2_e2v):
    f32 = jnp.float32
    i32 = jnp.int32
    vidx = edge_index[0]
    eidx = edge_index[1]
    gpad = jnp.zeros((_EP - _E,), i32)
    spad = jnp.full((_EP - _E,), _TRASH, i32)
    v3 = jnp.concatenate([vidx, gpad]).reshape(_NS, _CHP, _K)
    e3 = jnp.concatenate([eidx, gpad]).reshape(_NS, _CHP, _K)
    vs3 = jnp.concatenate([vidx, spad]).reshape(_NS, _CHP, _K)
    es3 = jnp.concatenate([eidx, spad]).reshape(_NS, _CHP, _K)
    xp = jnp.zeros((_NPAD, 128), f32).at[:_N, :].set(x)
    w2v = jnp.zeros((128, 64), f32).at[:, :40].set(W2_v2e)
    w2e = jnp.zeros((64, 64), f32).at[:40, :40].set(W2_e2v)
    gam = bn1_gamma.reshape(1, 128)
    bet = bn1_beta.reshape(1, 128)
    z80 = jnp.zeros((_RSUB, 80), f32)
    z32 = jnp.zeros((_RSUB, 32), f32)

    tL, tR = _tc_matmul_ones(xp, W1_v2e)
    p1L, p1R = _sc_seg80(tL, tR, v3, es3, z80)
    yL, yR, ce = _tc_mean_relu_mm_ones(p1L, p1R, W1_e2v)
    p2L, p2R = _sc_seg80(yL, yR, e3, vs3, z80)
    qL, qR, cv = _tc_mean_relu_bn_mm(p2L, p2R, gam, bet, w2v)
    r1L, r1R = _sc_seg32(qL, qR, v3, es3, z32)
    sL, sR = _tc_mean_relu_mm64(r1L, r1R, ce, w2e)
    r2L, r2R = _sc_seg32(sL, sR, e3, vs3, z32)
    out = _tc_mean_logsoftmax(r2L, r2R, cv)
    return out[:_N, :40]


# speedup vs baseline: 7.3744x; 1.5666x over previous
import functools

import jax
import jax.numpy as jnp
import numpy as np
from jax import lax
from jax.experimental import pallas as pl
from jax.experimental.pallas import tpu as pltpu
from jax.experimental.pallas import tpu_sc as plsc

_N = 10000
_E = 320000
_NPAD = 10240
_NC = 2
_NS = 16
_K = 128
_CHP = 160
_CHB = 20
_EP = _NS * _CHP * _K
_TRASH = 10200
_RSUB = _NPAD // _NS
_R = 512
_BN_SCALE = float(1.0 / np.sqrt(1.0 + 1e-5))

_mesh = plsc.VectorSubcoreMesh(
    core_axis_name="c", subcore_axis_name="s", num_cores=_NC, num_subcores=_NS)


def _make_sc_segsum(W):
    @functools.partial(
        pl.kernel,
        out_type=[jax.ShapeDtypeStruct((_NPAD, W), jnp.float32),
                  jax.ShapeDtypeStruct((_NPAD, W), jnp.float32)],
        mesh=_mesh,
        compiler_params=pltpu.CompilerParams(use_tc_tiling_on_sc=False),
        scratch_types=[pltpu.VMEM((_CHB, _K), jnp.int32),
                       pltpu.VMEM((_CHB, _K), jnp.int32),
                       pltpu.VMEM((_K, W), jnp.float32),
                       pltpu.VMEM((_K, W), jnp.float32),
                       pltpu.VMEM_SHARED((_NPAD, W), jnp.float32),
                       pltpu.VMEM_SHARED((_NPAD, W), jnp.float32),
                       pltpu.SemaphoreType.DMA,
                       pltpu.SemaphoreType.DMA,
                       pltpu.SemaphoreType.DMA])
    def _seg(tl_hbm, tr_hbm, gidx_hbm, sidx_hbm, zrows_hbm, outl_hbm,
             outr_hbm, gbuf, sbuf, r0, r1, tsp, acc, g0, g1, zsem):
        cid = lax.axis_index("c")
        sid = lax.axis_index("s")
        rows = [r0, r1]
        gsem = [g0, g1]
        row0 = pl.multiple_of(sid * _RSUB, 8)
        zcopy = pltpu.async_copy(zrows_hbm, acc.at[pl.ds(row0, _RSUB)], zsem)

        @pl.when(cid == 0)
        def _():
            pltpu.sync_copy(tl_hbm.at[pl.ds(row0, _RSUB)],
                            tsp.at[pl.ds(row0, _RSUB)])

        @pl.when(cid == 1)
        def _():
            pltpu.sync_copy(tr_hbm.at[pl.ds(row0, _RSUB)],
                            tsp.at[pl.ds(row0, _RSUB)])
        zcopy.wait()
        plsc.subcore_barrier()

        def gather_start(j, b):
            pltpu.async_copy(tsp.at[gbuf.at[j]], rows[b], gsem[b])

        def gather_wait(j, b):
            pltpu.make_async_copy(tsp.at[gbuf.at[j]], rows[b], gsem[b]).wait()

        for h in range(_CHP // _CHB):
            pltpu.sync_copy(gidx_hbm.at[sid, pl.ds(h * _CHB, _CHB)], gbuf)
            pltpu.sync_copy(sidx_hbm.at[sid, pl.ds(h * _CHB, _CHB)], sbuf)
            gather_start(0, 0)
            gather_start(1, 1)

            def pair(i, carry):
                for b in range(2):
                    j = i * 2 + b
                    gather_wait(j, b)
                    pltpu.sync_copy(rows[b], acc.at[sbuf.at[j]], add=True)

                    @pl.when(i < _CHB // 2 - 1)
                    def _():
                        gather_start(j + 2, b)
                return carry
            lax.fori_loop(0, _CHB // 2, pair, 0)

        plsc.subcore_barrier()

        @pl.when(cid == 0)
        def _():
            pltpu.sync_copy(acc.at[pl.ds(row0, _RSUB)],
                            outl_hbm.at[pl.ds(row0, _RSUB)])

        @pl.when(cid == 1)
        def _():
            pltpu.sync_copy(acc.at[pl.ds(row0, _RSUB)],
                            outr_hbm.at[pl.ds(row0, _RSUB)])

    return _seg


_sc_seg80 = _make_sc_segsum(80)
_sc_seg32 = _make_sc_segsum(32)


def _ones_tail(r):
    col = lax.broadcasted_iota(jnp.int32, (r, 16), 1)
    return jnp.where(col == 0, 1.0, 0.0).astype(jnp.float32)


def _split_ones(d):
    zs = jnp.zeros((_R, 16), jnp.float32)
    return (jnp.concatenate([d[:, :64], _ones_tail(_R)], axis=1),
            jnp.concatenate([d[:, 64:128], zs], axis=1))


def _tc1_body(x_ref, w_ref, ol_ref, or_ref):
    d = jnp.dot(x_ref[...], w_ref[...], preferred_element_type=jnp.float32)
    ol_ref[...], or_ref[...] = _split_ones(d)


def _mean128(pl_ref, pr_ref):
    sl = pl_ref[...]
    cnt = jnp.sum(sl[:, 64:80], axis=1, keepdims=True)
    s = jnp.concatenate([sl[:, :64], pr_ref[...][:, :64]], axis=1)
    return s / jnp.maximum(cnt, 1.0), cnt


def _tc2_body(pl_ref, pr_ref, w_ref, ol_ref, or_ref, c_out):
    m, cnt = _mean128(pl_ref, pr_ref)
    c_out[...] = cnt
    yc = jnp.maximum(m, 0.0)
    d = jnp.dot(yc, w_ref[...], preferred_element_type=jnp.float32)
    ol_ref[...], or_ref[...] = _split_ones(d)


def _tc3_body(pl_ref, pr_ref, g_ref, b_ref, w_ref, ol_ref, or_ref, c_out):
    m, cnt = _mean128(pl_ref, pr_ref)
    c_out[...] = cnt
    xv = jnp.maximum(m, 0.0)
    xv = xv * (g_ref[...] * _BN_SCALE) + b_ref[...]
    d = jnp.dot(xv, w_ref[...], preferred_element_type=jnp.float32)
    ol_ref[...] = d[:, :32]
    or_ref[...] = d[:, 32:64]


def _tc4_body(rl_ref, rr_ref, c_ref, w_ref, ol_ref, or_ref):
    s = jnp.concatenate([rl_ref[...], rr_ref[...]], axis=1)
    yc = jnp.maximum(s / jnp.maximum(c_ref[...], 1.0), 0.0)
    d = jnp.dot(yc, w_ref[...], preferred_element_type=jnp.float32)
    ol_ref[...] = d[:, :32]
    or_ref[...] = d[:, 32:64]


def _tc5_body(rl_ref, rr_ref, c_ref, o_ref):
    s = jnp.concatenate([rl_ref[...], rr_ref[...]], axis=1)
    z = s / jnp.maximum(c_ref[...], 1.0)
    col = lax.broadcasted_iota(jnp.int32, (_R, 64), 1)
    valid = col < 40
    zm = jnp.where(valid, z, -jnp.inf)
    m = jnp.max(zm, axis=1, keepdims=True)
    ez = jnp.where(valid, jnp.exp(z - m), 0.0)
    lse = jnp.log(jnp.sum(ez, axis=1, keepdims=True))
    o_ref[...] = jnp.where(valid, z - m - lse, 0.0)


_GRID = (_NPAD // _R,)


def _blk(c):
    return pl.BlockSpec((_R, c), lambda i: (i, 0))


def _cblk():
    return pl.BlockSpec((_R, 1), lambda i: (i, 0))


def _wblk(cin, cout):
    return pl.BlockSpec((cin, cout), lambda i: (0, 0))


def _f32(*shape):
    return jax.ShapeDtypeStruct(shape, jnp.float32)


def _tc_matmul_ones(xp, w):
    return pl.pallas_call(
        _tc1_body, grid=_GRID,
        in_specs=[_blk(128), _wblk(128, 128)],
        out_specs=[_blk(80), _blk(80)],
        out_shape=[_f32(_NPAD, 80), _f32(_NPAD, 80)],
    )(xp, w)


def _tc_mean_relu_mm_ones(pL, pR, w):
    return pl.pallas_call(
        _tc2_body, grid=_GRID,
        in_specs=[_blk(80), _blk(80), _wblk(128, 128)],
        out_specs=[_blk(80), _blk(80), _cblk()],
        out_shape=[_f32(_NPAD, 80), _f32(_NPAD, 80), _f32(_NPAD, 1)],
    )(pL, pR, w)


def _tc_mean_relu_bn_mm(pL, pR, gam, bet, w):
    return pl.pallas_call(
        _tc3_body, grid=_GRID,
        in_specs=[_blk(80), _blk(80),
                  pl.BlockSpec((1, 128), lambda i: (0, 0)),
                  pl.BlockSpec((1, 128), lambda i: (0, 0)),
                  _wblk(128, 64)],
        out_specs=[_blk(32), _blk(32), _cblk()],
        out_shape=[_f32(_NPAD, 32), _f32(_NPAD, 32), _f32(_NPAD, 1)],
    )(pL, pR, gam, bet, w)


def _tc_mean_relu_mm64(rL, rR, cnt, w):
    return pl.pallas_call(
        _tc4_body, grid=_GRID,
        in_specs=[_blk(32), _blk(32), _cblk(), _wblk(64, 64)],
        out_specs=[_blk(32), _blk(32)],
        out_shape=[_f32(_NPAD, 32), _f32(_NPAD, 32)],
    )(rL, rR, cnt, w)


def _tc_mean_logsoftmax(rL, rR, cnt):
    return pl.pallas_call(
        _tc5_body, grid=_GRID,
        in_specs=[_blk(32), _blk(32), _cblk()],
        out_specs=_blk(64),
        out_shape=_f32(_NPAD, 64),
    )(rL, rR, cnt)


def kernel(x, edge_index, W1_v2e, W1_e2v, bn1_gamma, bn1_beta, W2_v2e, W2_e2v):
    f32 = jnp.float32
    i32 = jnp.int32
    vidx = edge_index[0]
    eidx = edge_index[1]
    gpad = jnp.zeros((_EP - _E,), i32)
    spad = jnp.full((_EP - _E,), _TRASH, i32)
    v3 = jnp.concatenate([vidx, gpad]).reshape(_NS, _CHP, _K)
    e3 = jnp.concatenate([eidx, gpad]).reshape(_NS, _CHP, _K)
    vs3 = jnp.concatenate([vidx, spad]).reshape(_NS, _CHP, _K)
    es3 = jnp.concatenate([eidx, spad]).reshape(_NS, _CHP, _K)
    xp = jnp.zeros((_NPAD, 128), f32).at[:_N, :].set(x)
    w2v = jnp.zeros((128, 64), f32).at[:, :40].set(W2_v2e)
    w2e = jnp.zeros((64, 64), f32).at[:40, :40].set(W2_e2v)
    gam = bn1_gamma.reshape(1, 128)
    bet = bn1_beta.reshape(1, 128)
    z80 = jnp.zeros((_RSUB, 80), f32)
    z32 = jnp.zeros((_RSUB, 32), f32)

    tL, tR = _tc_matmul_ones(xp, W1_v2e)
    p1L, p1R = _sc_seg80(tL, tR, v3, es3, z80)
    yL, yR, ce = _tc_mean_relu_mm_ones(p1L, p1R, W1_e2v)
    p2L, p2R = _sc_seg80(yL, yR, e3, vs3, z80)
    qL, qR, cv = _tc_mean_relu_bn_mm(p2L, p2R, gam, bet, w2v)
    r1L, r1R = _sc_seg32(qL, qR, v3, es3, z32)
    sL, sR = _tc_mean_relu_mm64(r1L, r1R, ce, w2e)
    r2L, r2R = _sc_seg32(sL, sR, e3, vs3, z32)
    out = _tc_mean_logsoftmax(r2L, r2R, cv)
    return out[:_N, :40]


# speedup vs baseline: 7.8324x; 1.0621x over previous
import functools

import jax
import jax.numpy as jnp
import numpy as np
from jax import lax
from jax.experimental import pallas as pl
from jax.experimental.pallas import tpu as pltpu
from jax.experimental.pallas import tpu_sc as plsc

_N = 10000
_E = 320000
_NPAD = 10240
_NC = 2
_NS = 16
_K = 128
_CHP = 160
_CHB = 20
_EP = _NS * _CHP * _K
_TRASH = 10200
_RSUB = _NPAD // _NS
_R = 2048
_BN_SCALE = float(1.0 / np.sqrt(1.0 + 1e-5))

_mesh = plsc.VectorSubcoreMesh(
    core_axis_name="c", subcore_axis_name="s", num_cores=_NC, num_subcores=_NS)


def _make_sc_segsum(W):
    @functools.partial(
        pl.kernel,
        out_type=[jax.ShapeDtypeStruct((_NPAD, W), jnp.float32),
                  jax.ShapeDtypeStruct((_NPAD, W), jnp.float32)],
        mesh=_mesh,
        compiler_params=pltpu.CompilerParams(use_tc_tiling_on_sc=False),
        scratch_types=[pltpu.VMEM((_CHB, _K), jnp.int32),
                       pltpu.VMEM((_CHB, _K), jnp.int32),
                       pltpu.VMEM((_K, W), jnp.float32),
                       pltpu.VMEM((_K, W), jnp.float32),
                       pltpu.VMEM_SHARED((_NPAD, W), jnp.float32),
                       pltpu.VMEM_SHARED((_NPAD, W), jnp.float32),
                       pltpu.SemaphoreType.DMA,
                       pltpu.SemaphoreType.DMA,
                       pltpu.SemaphoreType.DMA])
    def _seg(tl_hbm, tr_hbm, gidx_hbm, sidx_hbm, zrows_hbm, outl_hbm,
             outr_hbm, gbuf, sbuf, r0, r1, tsp, acc, g0, g1, zsem):
        cid = lax.axis_index("c")
        sid = lax.axis_index("s")
        rows = [r0, r1]
        gsem = [g0, g1]
        row0 = pl.multiple_of(sid * _RSUB, 8)
        zcopy = pltpu.async_copy(zrows_hbm, acc.at[pl.ds(row0, _RSUB)], zsem)

        @pl.when(cid == 0)
        def _():
            pltpu.sync_copy(tl_hbm.at[pl.ds(row0, _RSUB)],
                            tsp.at[pl.ds(row0, _RSUB)])

        @pl.when(cid == 1)
        def _():
            pltpu.sync_copy(tr_hbm.at[pl.ds(row0, _RSUB)],
                            tsp.at[pl.ds(row0, _RSUB)])
        zcopy.wait()
        plsc.subcore_barrier()

        def gather_start(j, b):
            pltpu.async_copy(tsp.at[gbuf.at[j]], rows[b], gsem[b])

        def gather_wait(j, b):
            pltpu.make_async_copy(tsp.at[gbuf.at[j]], rows[b], gsem[b]).wait()

        for h in range(_CHP // _CHB):
            pltpu.sync_copy(gidx_hbm.at[sid, pl.ds(h * _CHB, _CHB)], gbuf)
            pltpu.sync_copy(sidx_hbm.at[sid, pl.ds(h * _CHB, _CHB)], sbuf)
            gather_start(0, 0)
            gather_start(1, 1)

            def pair(i, carry):
                for b in range(2):
                    j = i * 2 + b
                    gather_wait(j, b)
                    pltpu.sync_copy(rows[b], acc.at[sbuf.at[j]], add=True)

                    @pl.when(i < _CHB // 2 - 1)
                    def _():
                        gather_start(j + 2, b)
                return carry
            lax.fori_loop(0, _CHB // 2, pair, 0)

        plsc.subcore_barrier()

        @pl.when(cid == 0)
        def _():
            pltpu.sync_copy(acc.at[pl.ds(row0, _RSUB)],
                            outl_hbm.at[pl.ds(row0, _RSUB)])

        @pl.when(cid == 1)
        def _():
            pltpu.sync_copy(acc.at[pl.ds(row0, _RSUB)],
                            outr_hbm.at[pl.ds(row0, _RSUB)])

    return _seg


_sc_seg80 = _make_sc_segsum(80)
_sc_seg32 = _make_sc_segsum(32)


def _ones_tail(r):
    col = lax.broadcasted_iota(jnp.int32, (r, 16), 1)
    return jnp.where(col == 0, 1.0, 0.0).astype(jnp.float32)


def _split_ones(d):
    zs = jnp.zeros((_R, 16), jnp.float32)
    return (jnp.concatenate([d[:, :64], _ones_tail(_R)], axis=1),
            jnp.concatenate([d[:, 64:128], zs], axis=1))


def _tc1_body(x_ref, w_ref, ol_ref, or_ref):
    d = jnp.dot(x_ref[...], w_ref[...], preferred_element_type=jnp.float32)
    ol_ref[...], or_ref[...] = _split_ones(d)


def _mean128(pl_ref, pr_ref):
    sl = pl_ref[...]
    cnt = jnp.sum(sl[:, 64:80], axis=1, keepdims=True)
    s = jnp.concatenate([sl[:, :64], pr_ref[...][:, :64]], axis=1)
    return s / jnp.maximum(cnt, 1.0), cnt


def _tc2_body(pl_ref, pr_ref, w_ref, ol_ref, or_ref, c_out):
    m, cnt = _mean128(pl_ref, pr_ref)
    c_out[...] = cnt
    yc = jnp.maximum(m, 0.0)
    d = jnp.dot(yc, w_ref[...], preferred_element_type=jnp.float32)
    ol_ref[...], or_ref[...] = _split_ones(d)


def _tc3_body(pl_ref, pr_ref, g_ref, b_ref, w_ref, ol_ref, or_ref, c_out):
    m, cnt = _mean128(pl_ref, pr_ref)
    c_out[...] = cnt
    xv = jnp.maximum(m, 0.0)
    xv = xv * (g_ref[...] * _BN_SCALE) + b_ref[...]
    d = jnp.dot(xv, w_ref[...], preferred_element_type=jnp.float32)
    ol_ref[...] = d[:, :32]
    or_ref[...] = d[:, 32:64]


def _tc4_body(rl_ref, rr_ref, c_ref, w_ref, ol_ref, or_ref):
    s = jnp.concatenate([rl_ref[...], rr_ref[...]], axis=1)
    yc = jnp.maximum(s / jnp.maximum(c_ref[...], 1.0), 0.0)
    d = jnp.dot(yc, w_ref[...], preferred_element_type=jnp.float32)
    ol_ref[...] = d[:, :32]
    or_ref[...] = d[:, 32:64]


def _tc5_body(rl_ref, rr_ref, c_ref, o_ref):
    s = jnp.concatenate([rl_ref[...], rr_ref[...]], axis=1)
    z = s / jnp.maximum(c_ref[...], 1.0)
    col = lax.broadcasted_iota(jnp.int32, (_R, 64), 1)
    valid = col < 40
    zm = jnp.where(valid, z, -jnp.inf)
    m = jnp.max(zm, axis=1, keepdims=True)
    ez = jnp.where(valid, jnp.exp(z - m), 0.0)
    lse = jnp.log(jnp.sum(ez, axis=1, keepdims=True))
    o_ref[...] = jnp.where(valid, z - m - lse, 0.0)


_GRID = (_NPAD // _R,)


def _blk(c):
    return pl.BlockSpec((_R, c), lambda i: (i, 0))


def _cblk():
    return pl.BlockSpec((_R, 1), lambda i: (i, 0))


def _wblk(cin, cout):
    return pl.BlockSpec((cin, cout), lambda i: (0, 0))


def _f32(*shape):
    return jax.ShapeDtypeStruct(shape, jnp.float32)


def _tc_matmul_ones(xp, w):
    return pl.pallas_call(
        _tc1_body, grid=_GRID,
        in_specs=[_blk(128), _wblk(128, 128)],
        out_specs=[_blk(80), _blk(80)],
        out_shape=[_f32(_NPAD, 80), _f32(_NPAD, 80)],
    )(xp, w)


def _tc_mean_relu_mm_ones(pL, pR, w):
    return pl.pallas_call(
        _tc2_body, grid=_GRID,
        in_specs=[_blk(80), _blk(80), _wblk(128, 128)],
        out_specs=[_blk(80), _blk(80), _cblk()],
        out_shape=[_f32(_NPAD, 80), _f32(_NPAD, 80), _f32(_NPAD, 1)],
    )(pL, pR, w)


def _tc_mean_relu_bn_mm(pL, pR, gam, bet, w):
    return pl.pallas_call(
        _tc3_body, grid=_GRID,
        in_specs=[_blk(80), _blk(80),
                  pl.BlockSpec((1, 128), lambda i: (0, 0)),
                  pl.BlockSpec((1, 128), lambda i: (0, 0)),
                  _wblk(128, 64)],
        out_specs=[_blk(32), _blk(32), _cblk()],
        out_shape=[_f32(_NPAD, 32), _f32(_NPAD, 32), _f32(_NPAD, 1)],
    )(pL, pR, gam, bet, w)


def _tc_mean_relu_mm64(rL, rR, cnt, w):
    return pl.pallas_call(
        _tc4_body, grid=_GRID,
        in_specs=[_blk(32), _blk(32), _cblk(), _wblk(64, 64)],
        out_specs=[_blk(32), _blk(32)],
        out_shape=[_f32(_NPAD, 32), _f32(_NPAD, 32)],
    )(rL, rR, cnt, w)


def _tc_mean_logsoftmax(rL, rR, cnt):
    return pl.pallas_call(
        _tc5_body, grid=_GRID,
        in_specs=[_blk(32), _blk(32), _cblk()],
        out_specs=_blk(64),
        out_shape=_f32(_NPAD, 64),
    )(rL, rR, cnt)


def kernel(x, edge_index, W1_v2e, W1_e2v, bn1_gamma, bn1_beta, W2_v2e, W2_e2v):
    f32 = jnp.float32
    i32 = jnp.int32
    vidx = edge_index[0]
    eidx = edge_index[1]
    gpad = jnp.zeros((_EP - _E,), i32)
    spad = jnp.full((_EP - _E,), _TRASH, i32)
    v3 = jnp.concatenate([vidx, gpad]).reshape(_NS, _CHP, _K)
    e3 = jnp.concatenate([eidx, gpad]).reshape(_NS, _CHP, _K)
    vs3 = jnp.concatenate([vidx, spad]).reshape(_NS, _CHP, _K)
    es3 = jnp.concatenate([eidx, spad]).reshape(_NS, _CHP, _K)
    xp = jnp.zeros((_NPAD, 128), f32).at[:_N, :].set(x)
    w2v = jnp.zeros((128, 64), f32).at[:, :40].set(W2_v2e)
    w2e = jnp.zeros((64, 64), f32).at[:40, :40].set(W2_e2v)
    gam = bn1_gamma.reshape(1, 128)
    bet = bn1_beta.reshape(1, 128)
    z80 = jnp.zeros((_RSUB, 80), f32)
    z32 = jnp.zeros((_RSUB, 32), f32)

    tL, tR = _tc_matmul_ones(xp, W1_v2e)
    p1L, p1R = _sc_seg80(tL, tR, v3, es3, z80)
    yL, yR, ce = _tc_mean_relu_mm_ones(p1L, p1R, W1_e2v)
    p2L, p2R = _sc_seg80(yL, yR, e3, vs3, z80)
    qL, qR, cv = _tc_mean_relu_bn_mm(p2L, p2R, gam, bet, w2v)
    r1L, r1R = _sc_seg32(qL, qR, v3, es3, z32)
    sL, sR = _tc_mean_relu_mm64(r1L, r1R, ce, w2e)
    r2L, r2R = _sc_seg32(sL, sR, e3, vs3, z32)
    out = _tc_mean_logsoftmax(r2L, r2R, cv)
    return out[:_N, :40]


# speedup vs baseline: 8.3724x; 1.0689x over previous
import functools

import jax
import jax.numpy as jnp
import numpy as np
from jax import lax
from jax.experimental import pallas as pl
from jax.experimental.pallas import tpu as pltpu
from jax.experimental.pallas import tpu_sc as plsc

_N = 10000
_E = 320000
_NPAD = 10240
_NC = 2
_NS = 16
_K = 128
_CHP = 160
_CHB = 20
_EP = _NS * _CHP * _K
_TRASH = 10200
_RSUB = _NPAD // _NS
_R = 2048
_BN_SCALE = float(1.0 / np.sqrt(1.0 + 1e-5))

_mesh = plsc.VectorSubcoreMesh(
    core_axis_name="c", subcore_axis_name="s", num_cores=_NC, num_subcores=_NS)


def _make_sc_segsum(W):
    @functools.partial(
        pl.kernel,
        out_type=[jax.ShapeDtypeStruct((_NPAD, W), jnp.float32),
                  jax.ShapeDtypeStruct((_NPAD, W), jnp.float32)],
        mesh=_mesh,
        compiler_params=pltpu.CompilerParams(use_tc_tiling_on_sc=False),
        scratch_types=[pltpu.VMEM((_CHB, _K), jnp.int32),
                       pltpu.VMEM((_CHB, _K), jnp.int32),
                       pltpu.VMEM((_K, W), jnp.float32),
                       pltpu.VMEM((_K, W), jnp.float32),
                       pltpu.VMEM_SHARED((_NPAD, W), jnp.float32),
                       pltpu.VMEM_SHARED((_NPAD, W), jnp.float32),
                       pltpu.SemaphoreType.DMA,
                       pltpu.SemaphoreType.DMA,
                       pltpu.SemaphoreType.DMA])
    def _seg(tl_hbm, tr_hbm, gidx_hbm, sidx_hbm, zrows_hbm, outl_hbm,
             outr_hbm, gbuf, sbuf, r0, r1, tsp, acc, g0, g1, zsem):
        cid = lax.axis_index("c")
        sid = lax.axis_index("s")
        rows = [r0, r1]
        gsem = [g0, g1]
        row0 = pl.multiple_of(sid * _RSUB, 8)
        zcopy = pltpu.async_copy(zrows_hbm, acc.at[pl.ds(row0, _RSUB)], zsem)

        @pl.when(cid == 0)
        def _():
            pltpu.sync_copy(tl_hbm.at[pl.ds(row0, _RSUB)],
                            tsp.at[pl.ds(row0, _RSUB)])

        @pl.when(cid == 1)
        def _():
            pltpu.sync_copy(tr_hbm.at[pl.ds(row0, _RSUB)],
                            tsp.at[pl.ds(row0, _RSUB)])
        zcopy.wait()
        plsc.subcore_barrier()

        def gather_start(j, b):
            pltpu.async_copy(tsp.at[gbuf.at[j]], rows[b], gsem[b])

        def gather_wait(j, b):
            pltpu.make_async_copy(tsp.at[gbuf.at[j]], rows[b], gsem[b]).wait()

        for h in range(_CHP // _CHB):
            pltpu.sync_copy(gidx_hbm.at[sid, pl.ds(h * _CHB, _CHB)], gbuf)
            pltpu.sync_copy(sidx_hbm.at[sid, pl.ds(h * _CHB, _CHB)], sbuf)
            gather_start(0, 0)
            gather_start(1, 1)

            def pair(i, carry):
                for b in range(2):
                    j = i * 2 + b
                    gather_wait(j, b)
                    pltpu.sync_copy(rows[b], acc.at[sbuf.at[j]], add=True)

                    @pl.when(i < _CHB // 2 - 1)
                    def _():
                        gather_start(j + 2, b)
                return carry
            lax.fori_loop(0, _CHB // 2, pair, 0)

        plsc.subcore_barrier()

        @pl.when(cid == 0)
        def _():
            pltpu.sync_copy(acc.at[pl.ds(row0, _RSUB)],
                            outl_hbm.at[pl.ds(row0, _RSUB)])

        @pl.when(cid == 1)
        def _():
            pltpu.sync_copy(acc.at[pl.ds(row0, _RSUB)],
                            outr_hbm.at[pl.ds(row0, _RSUB)])

    return _seg


_sc_seg64 = _make_sc_segsum(64)
_sc_seg32 = _make_sc_segsum(32)


@functools.partial(
    pl.kernel,
    out_type=[jax.ShapeDtypeStruct((_NPAD, 16), jnp.float32),
              jax.ShapeDtypeStruct((_NPAD, 16), jnp.float32)],
    mesh=_mesh,
    compiler_params=pltpu.CompilerParams(use_tc_tiling_on_sc=False),
    scratch_types=[pltpu.VMEM((_CHB, _K), jnp.int32),
                   pltpu.VMEM((_K, 16), jnp.float32),
                   pltpu.VMEM_SHARED((_NPAD, 16), jnp.float32),
                   pltpu.SemaphoreType.DMA])
def _sc_counts(eidx_hbm, vidx_hbm, ones_hbm, zrows_hbm, oute_hbm, outv_hbm,
               sbuf, ones_sp, acc, zsem):
    cid = lax.axis_index("c")
    sid = lax.axis_index("s")
    row0 = pl.multiple_of(sid * _RSUB, 8)
    zcopy = pltpu.async_copy(zrows_hbm, acc.at[pl.ds(row0, _RSUB)], zsem)
    pltpu.sync_copy(ones_hbm, ones_sp)
    zcopy.wait()
    plsc.subcore_barrier()

    for h in range(_CHP // _CHB):
        @pl.when(cid == 0)
        def _():
            pltpu.sync_copy(eidx_hbm.at[sid, pl.ds(h * _CHB, _CHB)], sbuf)

        @pl.when(cid == 1)
        def _():
            pltpu.sync_copy(vidx_hbm.at[sid, pl.ds(h * _CHB, _CHB)], sbuf)

        def chunk(j, carry):
            pltpu.sync_copy(ones_sp, acc.at[sbuf.at[j]], add=True)
            return carry
        lax.fori_loop(0, _CHB, chunk, 0)

    plsc.subcore_barrier()

    @pl.when(cid == 0)
    def _():
        pltpu.sync_copy(acc.at[pl.ds(row0, _RSUB)],
                        oute_hbm.at[pl.ds(row0, _RSUB)])

    @pl.when(cid == 1)
    def _():
        pltpu.sync_copy(acc.at[pl.ds(row0, _RSUB)],
                        outv_hbm.at[pl.ds(row0, _RSUB)])


def _tc1_body(x_ref, w_ref, ol_ref, or_ref):
    d = jnp.dot(x_ref[...], w_ref[...], preferred_element_type=jnp.float32)
    ol_ref[...] = d[:, :64]
    or_ref[...] = d[:, 64:]


def _mean(pl_ref, pr_ref, c_ref):
    cnt = c_ref[...][:, :1]
    s = jnp.concatenate([pl_ref[...], pr_ref[...]], axis=1)
    return s / jnp.maximum(cnt, 1.0)


def _tc2_body(pl_ref, pr_ref, c_ref, w_ref, ol_ref, or_ref):
    yc = jnp.maximum(_mean(pl_ref, pr_ref, c_ref), 0.0)
    d = jnp.dot(yc, w_ref[...], preferred_element_type=jnp.float32)
    ol_ref[...] = d[:, :64]
    or_ref[...] = d[:, 64:]


def _tc3_body(pl_ref, pr_ref, c_ref, g_ref, b_ref, w_ref, ol_ref, or_ref):
    xv = jnp.maximum(_mean(pl_ref, pr_ref, c_ref), 0.0)
    xv = xv * (g_ref[...] * _BN_SCALE) + b_ref[...]
    d = jnp.dot(xv, w_ref[...], preferred_element_type=jnp.float32)
    ol_ref[...] = d[:, :32]
    or_ref[...] = d[:, 32:64]


def _tc4_body(rl_ref, rr_ref, c_ref, w_ref, ol_ref, or_ref):
    yc = jnp.maximum(_mean(rl_ref, rr_ref, c_ref), 0.0)
    d = jnp.dot(yc, w_ref[...], preferred_element_type=jnp.float32)
    ol_ref[...] = d[:, :32]
    or_ref[...] = d[:, 32:64]


def _tc5_body(rl_ref, rr_ref, c_ref, o_ref):
    z = _mean(rl_ref, rr_ref, c_ref)
    col = lax.broadcasted_iota(jnp.int32, (_R, 64), 1)
    valid = col < 40
    zm = jnp.where(valid, z, -jnp.inf)
    m = jnp.max(zm, axis=1, keepdims=True)
    ez = jnp.where(valid, jnp.exp(z - m), 0.0)
    lse = jnp.log(jnp.sum(ez, axis=1, keepdims=True))
    o_ref[...] = jnp.where(valid, z - m - lse, 0.0)


_GRID = (_NPAD // _R,)


def _blk(c):
    return pl.BlockSpec((_R, c), lambda i: (i, 0))


def _wblk(cin, cout):
    return pl.BlockSpec((cin, cout), lambda i: (0, 0))


def _f32(*shape):
    return jax.ShapeDtypeStruct(shape, jnp.float32)


def _tc_matmul(xp, w):
    return pl.pallas_call(
        _tc1_body, grid=_GRID,
        in_specs=[_blk(128), _wblk(128, 128)],
        out_specs=[_blk(64), _blk(64)],
        out_shape=[_f32(_NPAD, 64), _f32(_NPAD, 64)],
    )(xp, w)


def _tc_mean_relu_mm(pL, pR, cnt, w):
    return pl.pallas_call(
        _tc2_body, grid=_GRID,
        in_specs=[_blk(64), _blk(64), _blk(16), _wblk(128, 128)],
        out_specs=[_blk(64), _blk(64)],
        out_shape=[_f32(_NPAD, 64), _f32(_NPAD, 64)],
    )(pL, pR, cnt, w)


def _tc_mean_relu_bn_mm(pL, pR, cnt, gam, bet, w):
    return pl.pallas_call(
        _tc3_body, grid=_GRID,
        in_specs=[_blk(64), _blk(64), _blk(16),
                  pl.BlockSpec((1, 128), lambda i: (0, 0)),
                  pl.BlockSpec((1, 128), lambda i: (0, 0)),
                  _wblk(128, 64)],
        out_specs=[_blk(32), _blk(32)],
        out_shape=[_f32(_NPAD, 32), _f32(_NPAD, 32)],
    )(pL, pR, cnt, gam, bet, w)


def _tc_mean_relu_mm64(rL, rR, cnt, w):
    return pl.pallas_call(
        _tc4_body, grid=_GRID,
        in_specs=[_blk(32), _blk(32), _blk(16), _wblk(64, 64)],
        out_specs=[_blk(32), _blk(32)],
        out_shape=[_f32(_NPAD, 32), _f32(_NPAD, 32)],
    )(rL, rR, cnt, w)


def _tc_mean_logsoftmax(rL, rR, cnt):
    return pl.pallas_call(
        _tc5_body, grid=_GRID,
        in_specs=[_blk(32), _blk(32), _blk(16)],
        out_specs=_blk(64),
        out_shape=_f32(_NPAD, 64),
    )(rL, rR, cnt)


def kernel(x, edge_index, W1_v2e, W1_e2v, bn1_gamma, bn1_beta, W2_v2e, W2_e2v):
    f32 = jnp.float32
    i32 = jnp.int32
    vidx = edge_index[0]
    eidx = edge_index[1]
    gpad = jnp.zeros((_EP - _E,), i32)
    spad = jnp.full((_EP - _E,), _TRASH, i32)
    v3 = jnp.concatenate([vidx, gpad]).reshape(_NS, _CHP, _K)
    e3 = jnp.concatenate([eidx, gpad]).reshape(_NS, _CHP, _K)
    vs3 = jnp.concatenate([vidx, spad]).reshape(_NS, _CHP, _K)
    es3 = jnp.concatenate([eidx, spad]).reshape(_NS, _CHP, _K)
    xp = jnp.zeros((_NPAD, 128), f32).at[:_N, :].set(x)
    w2v = jnp.zeros((128, 64), f32).at[:, :40].set(W2_v2e)
    w2e = jnp.zeros((64, 64), f32).at[:40, :40].set(W2_e2v)
    gam = bn1_gamma.reshape(1, 128)
    bet = bn1_beta.reshape(1, 128)
    ones16 = jnp.zeros((_K, 16), f32).at[:, 0].set(1.0)
    z16 = jnp.zeros((_RSUB, 16), f32)
    z64 = jnp.zeros((_RSUB, 64), f32)
    z32 = jnp.zeros((_RSUB, 32), f32)

    ce, cv = _sc_counts(es3, vs3, ones16, z16)
    tL, tR = _tc_matmul(xp, W1_v2e)
    p1L, p1R = _sc_seg64(tL, tR, v3, es3, z64)
    yL, yR = _tc_mean_relu_mm(p1L, p1R, ce, W1_e2v)
    p2L, p2R = _sc_seg64(yL, yR, e3, vs3, z64)
    qL, qR = _tc_mean_relu_bn_mm(p2L, p2R, cv, gam, bet, w2v)
    r1L, r1R = _sc_seg32(qL, qR, v3, es3, z32)
    sL, sR = _tc_mean_relu_mm64(r1L, r1R, ce, w2e)
    r2L, r2R = _sc_seg32(sL, sR, e3, vs3, z32)
    out = _tc_mean_logsoftmax(r2L, r2R, cv)
    return out[:_N, :40]


# speedup vs baseline: 8.6866x; 1.0375x over previous
import functools

import jax
import jax.numpy as jnp
import numpy as np
from jax import lax
from jax.experimental import pallas as pl
from jax.experimental.pallas import tpu as pltpu
from jax.experimental.pallas import tpu_sc as plsc

_N = 10000
_E = 320000
_NPAD = 10240
_NC = 2
_NS = 16
_K = 128
_CHP = 160
_CHB = 20
_EP = _NS * _CHP * _K
_TRASH = 10200
_RSUB = _NPAD // _NS
_R = 2048
_BN_SCALE = float(1.0 / np.sqrt(1.0 + 1e-5))

_mesh = plsc.VectorSubcoreMesh(
    core_axis_name="c", subcore_axis_name="s", num_cores=_NC, num_subcores=_NS)


def _make_sc_segsum(W):
    @functools.partial(
        pl.kernel,
        out_type=[jax.ShapeDtypeStruct((_NPAD, W), jnp.float32),
                  jax.ShapeDtypeStruct((_NPAD, W), jnp.float32)],
        mesh=_mesh,
        compiler_params=pltpu.CompilerParams(use_tc_tiling_on_sc=False),
        scratch_types=[pltpu.VMEM((_CHB, _K), jnp.int32),
                       pltpu.VMEM((_CHB, _K), jnp.int32),
                       pltpu.VMEM((_K, W), jnp.float32),
                       pltpu.VMEM((_K, W), jnp.float32),
                       pltpu.VMEM_SHARED((_NPAD, W), jnp.float32),
                       pltpu.VMEM_SHARED((_NPAD, W), jnp.float32),
                       pltpu.SemaphoreType.DMA,
                       pltpu.SemaphoreType.DMA,
                       pltpu.SemaphoreType.DMA])
    def _seg(tl_hbm, tr_hbm, gidx_hbm, sidx_hbm, zrows_hbm, outl_hbm,
             outr_hbm, gbuf, sbuf, r0, r1, tsp, acc, g0, g1, zsem):
        cid = lax.axis_index("c")
        sid = lax.axis_index("s")
        rows = [r0, r1]
        gsem = [g0, g1]
        row0 = pl.multiple_of(sid * _RSUB, 8)
        zcopy = pltpu.async_copy(zrows_hbm, acc.at[pl.ds(row0, _RSUB)], zsem)

        @pl.when(cid == 0)
        def _():
            pltpu.sync_copy(tl_hbm.at[pl.ds(row0, _RSUB)],
                            tsp.at[pl.ds(row0, _RSUB)])

        @pl.when(cid == 1)
        def _():
            pltpu.sync_copy(tr_hbm.at[pl.ds(row0, _RSUB)],
                            tsp.at[pl.ds(row0, _RSUB)])
        zcopy.wait()
        plsc.subcore_barrier()

        def gather_start(j, b):
            pltpu.async_copy(tsp.at[gbuf.at[j]], rows[b], gsem[b])

        def gather_wait(j, b):
            pltpu.make_async_copy(tsp.at[gbuf.at[j]], rows[b], gsem[b]).wait()

        for h in range(_CHP // _CHB):
            pltpu.sync_copy(gidx_hbm.at[sid, pl.ds(h * _CHB, _CHB)], gbuf)
            pltpu.sync_copy(sidx_hbm.at[sid, pl.ds(h * _CHB, _CHB)], sbuf)
            gather_start(0, 0)
            gather_start(1, 1)

            def pair(i, carry):
                for b in range(2):
                    j = i * 2 + b
                    gather_wait(j, b)
                    pltpu.sync_copy(rows[b], acc.at[sbuf.at[j]], add=True)

                    @pl.when(i < _CHB // 2 - 1)
                    def _():
                        gather_start(j + 2, b)
                return carry
            lax.fori_loop(0, _CHB // 2, pair, 0)

        plsc.subcore_barrier()

        @pl.when(cid == 0)
        def _():
            pltpu.sync_copy(acc.at[pl.ds(row0, _RSUB)],
                            outl_hbm.at[pl.ds(row0, _RSUB)])

        @pl.when(cid == 1)
        def _():
            pltpu.sync_copy(acc.at[pl.ds(row0, _RSUB)],
                            outr_hbm.at[pl.ds(row0, _RSUB)])

    return _seg


_sc_seg64 = _make_sc_segsum(64)
_sc_seg24 = _make_sc_segsum(24)


@functools.partial(
    pl.kernel,
    out_type=[jax.ShapeDtypeStruct((_NPAD, 16), jnp.float32),
              jax.ShapeDtypeStruct((_NPAD, 16), jnp.float32)],
    mesh=_mesh,
    compiler_params=pltpu.CompilerParams(use_tc_tiling_on_sc=False),
    scratch_types=[pltpu.VMEM((_CHB, _K), jnp.int32),
                   pltpu.VMEM((_K, 16), jnp.float32),
                   pltpu.VMEM_SHARED((_NPAD, 16), jnp.float32),
                   pltpu.SemaphoreType.DMA])
def _sc_counts(eidx_hbm, vidx_hbm, ones_hbm, zrows_hbm, oute_hbm, outv_hbm,
               sbuf, ones_sp, acc, zsem):
    cid = lax.axis_index("c")
    sid = lax.axis_index("s")
    row0 = pl.multiple_of(sid * _RSUB, 8)
    zcopy = pltpu.async_copy(zrows_hbm, acc.at[pl.ds(row0, _RSUB)], zsem)
    pltpu.sync_copy(ones_hbm, ones_sp)
    zcopy.wait()
    plsc.subcore_barrier()

    for h in range(_CHP // _CHB):
        @pl.when(cid == 0)
        def _():
            pltpu.sync_copy(eidx_hbm.at[sid, pl.ds(h * _CHB, _CHB)], sbuf)

        @pl.when(cid == 1)
        def _():
            pltpu.sync_copy(vidx_hbm.at[sid, pl.ds(h * _CHB, _CHB)], sbuf)

        def chunk(j, carry):
            pltpu.sync_copy(ones_sp, acc.at[sbuf.at[j]], add=True)
            return carry
        lax.fori_loop(0, _CHB, chunk, 0)

    plsc.subcore_barrier()

    @pl.when(cid == 0)
    def _():
        pltpu.sync_copy(acc.at[pl.ds(row0, _RSUB)],
                        oute_hbm.at[pl.ds(row0, _RSUB)])

    @pl.when(cid == 1)
    def _():
        pltpu.sync_copy(acc.at[pl.ds(row0, _RSUB)],
                        outv_hbm.at[pl.ds(row0, _RSUB)])


def _tc1_body(x_ref, w_ref, ol_ref, or_ref):
    d = jnp.dot(x_ref[...], w_ref[...], preferred_element_type=jnp.float32)
    ol_ref[...] = d[:, :64]
    or_ref[...] = d[:, 64:]


def _mean(pl_ref, pr_ref, c_ref):
    cnt = c_ref[...][:, :1]
    s = jnp.concatenate([pl_ref[...], pr_ref[...]], axis=1)
    return s / jnp.maximum(cnt, 1.0)


def _tc2_body(pl_ref, pr_ref, c_ref, w_ref, ol_ref, or_ref):
    yc = jnp.maximum(_mean(pl_ref, pr_ref, c_ref), 0.0)
    d = jnp.dot(yc, w_ref[...], preferred_element_type=jnp.float32)
    ol_ref[...] = d[:, :64]
    or_ref[...] = d[:, 64:]


def _tc3_body(pl_ref, pr_ref, c_ref, g_ref, b_ref, w_ref, ol_ref, or_ref):
    xv = jnp.maximum(_mean(pl_ref, pr_ref, c_ref), 0.0)
    xv = xv * (g_ref[...] * _BN_SCALE) + b_ref[...]
    d = jnp.dot(xv, w_ref[...], preferred_element_type=jnp.float32)
    ol_ref[...] = d[:, :24]
    or_ref[...] = d[:, 24:48]


def _tc4_body(rl_ref, rr_ref, c_ref, w_ref, ol_ref, or_ref):
    yc = jnp.maximum(_mean(rl_ref, rr_ref, c_ref), 0.0)
    d = jnp.dot(yc, w_ref[...], preferred_element_type=jnp.float32)
    ol_ref[...] = d[:, :24]
    or_ref[...] = d[:, 24:48]


def _tc5_body(rl_ref, rr_ref, c_ref, o_ref):
    z = _mean(rl_ref, rr_ref, c_ref)
    col = lax.broadcasted_iota(jnp.int32, (_R, 48), 1)
    valid = col < 40
    zm = jnp.where(valid, z, -jnp.inf)
    m = jnp.max(zm, axis=1, keepdims=True)
    ez = jnp.where(valid, jnp.exp(z - m), 0.0)
    lse = jnp.log(jnp.sum(ez, axis=1, keepdims=True))
    o_ref[...] = jnp.where(valid, z - m - lse, 0.0)


_GRID = (_NPAD // _R,)


def _blk(c):
    return pl.BlockSpec((_R, c), lambda i: (i, 0))


def _wblk(cin, cout):
    return pl.BlockSpec((cin, cout), lambda i: (0, 0))


def _f32(*shape):
    return jax.ShapeDtypeStruct(shape, jnp.float32)


def _tc_matmul(xp, w):
    return pl.pallas_call(
        _tc1_body, grid=_GRID,
        in_specs=[_blk(128), _wblk(128, 128)],
        out_specs=[_blk(64), _blk(64)],
        out_shape=[_f32(_NPAD, 64), _f32(_NPAD, 64)],
    )(xp, w)


def _tc_mean_relu_mm(pL, pR, cnt, w):
    return pl.pallas_call(
        _tc2_body, grid=_GRID,
        in_specs=[_blk(64), _blk(64), _blk(16), _wblk(128, 128)],
        out_specs=[_blk(64), _blk(64)],
        out_shape=[_f32(_NPAD, 64), _f32(_NPAD, 64)],
    )(pL, pR, cnt, w)


def _tc_mean_relu_bn_mm(pL, pR, cnt, gam, bet, w):
    return pl.pallas_call(
        _tc3_body, grid=_GRID,
        in_specs=[_blk(64), _blk(64), _blk(16),
                  pl.BlockSpec((1, 128), lambda i: (0, 0)),
                  pl.BlockSpec((1, 128), lambda i: (0, 0)),
                  _wblk(128, 48)],
        out_specs=[_blk(24), _blk(24)],
        out_shape=[_f32(_NPAD, 24), _f32(_NPAD, 24)],
    )(pL, pR, cnt, gam, bet, w)


def _tc_mean_relu_mm48(rL, rR, cnt, w):
    return pl.pallas_call(
        _tc4_body, grid=_GRID,
        in_specs=[_blk(24), _blk(24), _blk(16), _wblk(48, 48)],
        out_specs=[_blk(24), _blk(24)],
        out_shape=[_f32(_NPAD, 24), _f32(_NPAD, 24)],
    )(rL, rR, cnt, w)


def _tc_mean_logsoftmax(rL, rR, cnt):
    return pl.pallas_call(
        _tc5_body, grid=_GRID,
        in_specs=[_blk(24), _blk(24), _blk(16)],
        out_specs=_blk(48),
        out_shape=_f32(_NPAD, 48),
    )(rL, rR, cnt)


def kernel(x, edge_index, W1_v2e, W1_e2v, bn1_gamma, bn1_beta, W2_v2e, W2_e2v):
    f32 = jnp.float32
    i32 = jnp.int32
    vidx = edge_index[0]
    eidx = edge_index[1]
    gpad = jnp.zeros((_EP - _E,), i32)
    spad = jnp.full((_EP - _E,), _TRASH, i32)
    v3 = jnp.concatenate([vidx, gpad]).reshape(_NS, _CHP, _K)
    e3 = jnp.concatenate([eidx, gpad]).reshape(_NS, _CHP, _K)
    vs3 = jnp.concatenate([vidx, spad]).reshape(_NS, _CHP, _K)
    es3 = jnp.concatenate([eidx, spad]).reshape(_NS, _CHP, _K)
    xp = jnp.zeros((_NPAD, 128), f32).at[:_N, :].set(x)
    w2v = jnp.zeros((128, 48), f32).at[:, :40].set(W2_v2e)
    w2e = jnp.zeros((48, 48), f32).at[:40, :40].set(W2_e2v)
    gam = bn1_gamma.reshape(1, 128)
    bet = bn1_beta.reshape(1, 128)
    ones16 = jnp.zeros((_K, 16), f32).at[:, 0].set(1.0)
    z16 = jnp.zeros((_RSUB, 16), f32)
    z64 = jnp.zeros((_RSUB, 64), f32)
    z24 = jnp.zeros((_RSUB, 24), f32)

    ce, cv = _sc_counts(es3, vs3, ones16, z16)
    tL, tR = _tc_matmul(xp, W1_v2e)
    p1L, p1R = _sc_seg64(tL, tR, v3, es3, z64)
    yL, yR = _tc_mean_relu_mm(p1L, p1R, ce, W1_e2v)
    p2L, p2R = _sc_seg64(yL, yR, e3, vs3, z64)
    qL, qR = _tc_mean_relu_bn_mm(p2L, p2R, cv, gam, bet, w2v)
    r1L, r1R = _sc_seg24(qL, qR, v3, es3, z24)
    sL, sR = _tc_mean_relu_mm48(r1L, r1R, ce, w2e)
    r2L, r2R = _sc_seg24(sL, sR, e3, vs3, z24)
    out = _tc_mean_logsoftmax(r2L, r2R, cv)
    return out[:_N, :40]
